# baseline (speedup 1.0000x reference)
import functools

import jax
import jax.numpy as jnp
from jax import lax
from jax.experimental import pallas as pl
from jax.experimental.pallas import tpu as pltpu
from jax.experimental.pallas import tpu_sc as plsc

D = 64
A = 4096
B = 200
CHUNK = 128
SCALE = float(D) ** -0.5
NBUF = 4


def _make_sc_kernel():
    info = plsc.get_sparse_core_info()
    nc, ns = info.num_cores, info.num_subcores
    nw = nc * ns

    mesh = plsc.VectorSubcoreMesh(core_axis_name="c", subcore_axis_name="s")

    scratch = [
        pltpu.VMEM((B, CHUNK), jnp.int32),
        pltpu.VMEM((NBUF, CHUNK, D), jnp.float32),
        pltpu.VMEM((NBUF, D, CHUNK), jnp.float32),
    ] + [pltpu.SemaphoreType.DMA] * (2 * NBUF)

    @functools.partial(
        pl.kernel,
        out_type=jax.ShapeDtypeStruct((B * 8 * nw * 8, CHUNK), jnp.float32),
        mesh=mesh,
        scratch_types=scratch,
        compiler_params=pltpu.CompilerParams(use_tc_tiling_on_sc=False,
                                             needs_layout_passes=False),
    )
    def emb_kernel(idx_hbm, table_hbm, out_hbm, idx_v, in_v, tr_v, *sems):
        gsem = sems[:NBUF]
        ssem = sems[NBUF:]
        wid = lax.axis_index("s") * nc + lax.axis_index("c")
        pltpu.sync_copy(idx_hbm.at[wid], idx_v)

        lanes = lax.iota(jnp.int32, 16)
        dvecs = [lanes + 16 * g for g in range(D // 16)]

        def start_gather(bb, c):
            pltpu.async_copy(table_hbm.at[idx_v.at[c]], in_v.at[bb], gsem[bb])

        def start_store(bb, c):
            for dt in range(8):
                base = ((c * 8 + dt) * nw + wid) * 8
                pltpu.async_copy(tr_v.at[bb, pl.ds(dt * 8, 8)],
                                 out_hbm.at[pl.ds(base, 8)], ssem[bb])

        def wait_gather(bb):
            pltpu.make_async_copy(table_hbm.at[idx_v.at[0]], in_v.at[bb],
                                  gsem[bb]).wait()

        def wait_store(bb):
            pltpu.make_async_copy(tr_v.at[bb],
                                  out_hbm.at[pl.ds(0, D)], ssem[bb]).wait()

        def transpose(bb):
            dst = tr_v.at[bb]

            def row(j, carry):
                jvec = jnp.broadcast_to(j, (16,)).astype(jnp.int32)
                vals = [in_v[bb, j, pl.ds(g * 16, 16)] * SCALE
                        for g in range(D // 16)]
                for g in range(D // 16):
                    plsc.store_scatter(dst, [dvecs[g], jvec], vals[g])
                return carry

            lax.fori_loop(0, CHUNK, row, 0, unroll=4)

        for bb in range(NBUF):
            start_gather(bb, bb)
        for bb in range(NBUF):
            wait_gather(bb)
            transpose(bb)
            start_store(bb, bb)
            start_gather(bb, NBUF + bb)

        nsteps = B // NBUF

        def step(g0, carry):
            for bb in range(NBUF):
                c = g0 * NBUF + bb
                wait_gather(bb)
                wait_store(bb)
                transpose(bb)
                start_store(bb, c)
                start_gather(bb, c + NBUF)
            return carry

        lax.fori_loop(1, nsteps - 1, step, 0)

        for bb in range(NBUF):
            c = (nsteps - 1) * NBUF + bb
            wait_gather(bb)
            wait_store(bb)
            transpose(bb)
            start_store(bb, c)
        for bb in range(NBUF):
            wait_store(bb)

    return emb_kernel


_emb = _make_sc_kernel()


@jax.jit
def kernel(x, table):
    idx3 = (x.T.reshape(B, 32, CHUNK).transpose(1, 0, 2)
            .astype(jnp.int32))
    outp = _emb(idx3, table)
    out = (outp.reshape(B, 8, 32, 8, CHUNK)
           .transpose(2, 4, 0, 1, 3)
           .reshape(A, B, D))
    return out


# speedup vs baseline: 2.1586x; 2.1586x over previous
import functools

import jax
import jax.numpy as jnp
from jax import lax
from jax.experimental import pallas as pl
from jax.experimental.pallas import tpu as pltpu
from jax.experimental.pallas import tpu_sc as plsc

D = 64
A = 4096
B = 200
CHUNK = 128
SCALE = float(D) ** -0.5
NBUF = 4


def _make_sc_kernel():
    info = plsc.get_sparse_core_info()
    nc, ns = info.num_cores, info.num_subcores
    nw = nc * ns

    mesh = plsc.VectorSubcoreMesh(core_axis_name="c", subcore_axis_name="s")

    scratch = [
        pltpu.VMEM((B, CHUNK), jnp.int32),
        pltpu.VMEM((NBUF, CHUNK, D), jnp.float32),
        pltpu.VMEM((NBUF, D, CHUNK), jnp.float32),
    ] + [pltpu.SemaphoreType.DMA] * (2 * NBUF)

    @functools.partial(
        pl.kernel,
        out_type=jax.ShapeDtypeStruct((B * 8 * nw * 8, CHUNK), jnp.float32),
        mesh=mesh,
        scratch_types=scratch,
        compiler_params=pltpu.CompilerParams(use_tc_tiling_on_sc=False,
                                             needs_layout_passes=False),
    )
    def emb_kernel(idx_hbm, table_hbm, out_hbm, idx_v, in_v, tr_v, *sems):
        gsem = sems[:NBUF]
        ssem = sems[NBUF:]
        wid = lax.axis_index("s") * nc + lax.axis_index("c")
        pltpu.sync_copy(idx_hbm.at[wid], idx_v)

        lanes = lax.iota(jnp.int32, 16)
        dvecs = [lanes + 16 * g for g in range(D // 16)]

        def start_gather(bb, c):
            pltpu.async_copy(table_hbm.at[idx_v.at[c]], in_v.at[bb], gsem[bb])

        def start_store(bb, c):
            for dt in range(8):
                base = ((c * 8 + dt) * nw + wid) * 8
                pltpu.async_copy(tr_v.at[bb, pl.ds(dt * 8, 8)],
                                 out_hbm.at[pl.ds(base, 8)], ssem[bb])

        def wait_gather(bb):
            pltpu.make_async_copy(table_hbm.at[idx_v.at[0]], in_v.at[bb],
                                  gsem[bb]).wait()

        def wait_store(bb):
            pltpu.make_async_copy(tr_v.at[bb],
                                  out_hbm.at[pl.ds(0, D)], ssem[bb]).wait()

        def transpose(bb):
            dst = tr_v.at[bb]

            @functools.partial(plsc.parallel_loop, 0, CHUNK, unroll=4)
            def row(j):
                jvec = jnp.broadcast_to(j, (16,)).astype(jnp.int32)
                vals = [in_v[bb, j, pl.ds(g * 16, 16)] * SCALE
                        for g in range(D // 16)]
                for g in range(D // 16):
                    plsc.store_scatter(dst, [dvecs[g], jvec], vals[g])

            v0 = tr_v[bb, 0, pl.ds(0, 16)]
            tr_v[bb, 0, pl.ds(0, 16)] = v0

        for bb in range(NBUF):
            start_gather(bb, bb)
        for bb in range(NBUF):
            wait_gather(bb)
            transpose(bb)
            start_store(bb, bb)
            start_gather(bb, NBUF + bb)

        nsteps = B // NBUF

        def step(g0, carry):
            for bb in range(NBUF):
                c = g0 * NBUF + bb
                wait_gather(bb)
                wait_store(bb)
                transpose(bb)
                start_store(bb, c)
                start_gather(bb, c + NBUF)
            return carry

        lax.fori_loop(1, nsteps - 1, step, 0)

        for bb in range(NBUF):
            c = (nsteps - 1) * NBUF + bb
            wait_gather(bb)
            wait_store(bb)
            transpose(bb)
            start_store(bb, c)
        for bb in range(NBUF):
            wait_store(bb)

    return emb_kernel


_emb = _make_sc_kernel()


@jax.jit
def kernel(x, table):
    idx3 = (x.T.reshape(B, 32, CHUNK).transpose(1, 0, 2)
            .astype(jnp.int32))
    outp = _emb(idx3, table)
    out = (outp.reshape(B, 8, 32, 8, CHUNK)
           .transpose(2, 4, 0, 1, 3)
           .reshape(A, B, D))
    return out
